# baseline (speedup 1.0000x reference)
import jax
import jax.numpy as jnp
from jax.experimental import pallas as pl
from jax.experimental.pallas import tpu as pltpu

NCLS = 17
FILT_K = 100
BLK = 512
D = 2048
LANES = 128
NEG = -1e30
IMAX = jnp.iinfo(jnp.int32).max


def _k1_body(x_ref, wc_ref, wm_ref, wcp_ref, bcp_ref,
             s_ref, ent_ref, y_ref, invn_ref, sc_ref):
    i = pl.program_id(0)

    s = sc_ref[...]
    logits = jnp.dot(s, wcp_ref[...].astype(jnp.bfloat16),
                     preferred_element_type=jnp.float32) + bcp_ref[...]
    m = jnp.max(logits, axis=1, keepdims=True)
    z = logits - m
    e = jnp.exp(z)
    se = jnp.sum(e, axis=1, keepdims=True)
    p = e / se
    ls = z - jnp.log(se)
    ent = -jnp.sum(p * ls, axis=1, keepdims=True)
    rows = jax.lax.broadcasted_iota(jnp.int32, (BLK, 1), 0)
    pad_row = jnp.logical_and(i == 1, rows >= NCLS)
    ent_ref[...] = jnp.where(pad_row, jnp.float32(jnp.inf), ent)
    lanes = jax.lax.broadcasted_iota(jnp.int32, (BLK, LANES), 1)
    y_ref[...] = jnp.min(jnp.where(logits == m, lanes, LANES), axis=1,
                         keepdims=True)
    sf = s.astype(jnp.float32)
    rn = jnp.sqrt(jnp.sum(sf * sf, axis=1, keepdims=True))
    invn_ref[...] = 1.0 / jnp.maximum(rn, 1e-12)

    s_new = jnp.dot(x_ref[...], wm_ref[...],
                    preferred_element_type=jnp.float32).astype(jnp.bfloat16)
    s_new = jnp.where(i == 0, wc_ref[...].astype(jnp.bfloat16), s_new)
    s_ref[...] = s_new
    sc_ref[...] = s_new


def _k2_body(ent_ref, y_ref, invn_ref, sc_ref):
    n = ent_ref.shape[0]
    key = jax.lax.bitcast_convert_type(jnp.maximum(ent_ref[...], 0.0),
                                       jnp.int32)
    lanes = jax.lax.broadcasted_iota(jnp.int32, (n, LANES), 1)
    yoh = y_ref[...] == lanes
    mk = jnp.where(yoh, key, IMAX)

    def body(it, lo):
        mid = lo + jnp.left_shift(jnp.int32(1), 30 - it)
        cnt = jnp.sum(jnp.where(mk < mid, 1, 0), axis=0, keepdims=True)
        return jnp.where(cnt >= FILT_K, lo, mid)

    lo = jax.lax.fori_loop(0, 31, body, jnp.zeros((1, LANES), jnp.int32))
    tau = jnp.sum(jnp.where(yoh, lo, 0), axis=1, keepdims=True)
    sc_ref[...] = jnp.where(key <= tau, invn_ref[...], 0.0)


def _k3_body(s_ref, y_ref, sc_ref, o_ref, w_ref, wbf_ref):
    i = pl.program_id(0)
    nacc = (pl.num_programs(0) + 1) // 2

    @pl.when(i == 0)
    def _():
        w_ref[...] = jnp.zeros_like(w_ref)

    @pl.when(i < nacc)
    def _():
        lanes = jax.lax.broadcasted_iota(jnp.int32, (BLK, LANES), 1)
        msel = jnp.where(y_ref[...] == lanes, sc_ref[...],
                         0.0).astype(jnp.bfloat16)
        w_ref[...] += jax.lax.dot_general(
            s_ref[...], msel, (((0,), (0,)), ((), ())),
            preferred_element_type=jnp.float32)

    @pl.when(i == nacc - 1)
    def _():
        w = w_ref[...]
        cn = jnp.sqrt(jnp.sum(w * w, axis=0, keepdims=True))
        wbf_ref[...] = (w / jnp.maximum(cn, 1e-12)).astype(jnp.bfloat16)

    @pl.when(i >= nacc)
    def _():
        o_ref[...] = jnp.dot(s_ref[...], wbf_ref[...],
                             preferred_element_type=jnp.float32)


def kernel(x, Wm, Wc, bc):
    b = x.shape[0]
    n = b + BLK
    nblk = n // BLK
    wc_pad = jnp.zeros((BLK, D), jnp.float32).at[:NCLS].set(Wc)
    wcp = jnp.zeros((D, LANES), jnp.float32).at[:, :NCLS].set(Wc.T)
    bcp = jnp.full((1, LANES), NEG, jnp.float32).at[0, :NCLS].set(bc)

    nxblk = b // BLK

    s, ent, y, invn = pl.pallas_call(
        _k1_body,
        grid=(nblk + 1,),
        in_specs=[
            pl.BlockSpec(
                (BLK, D),
                lambda i: (jnp.minimum(jnp.maximum(i - 1, 0), nxblk - 1), 0)),
            pl.BlockSpec((BLK, D), lambda i: (0, 0)),
            pl.BlockSpec((D, D), lambda i: (0, 0)),
            pl.BlockSpec((D, LANES), lambda i: (0, 0)),
            pl.BlockSpec((1, LANES), lambda i: (0, 0)),
        ],
        out_specs=[
            pl.BlockSpec((BLK, D), lambda i: (jnp.minimum(i, nblk - 1), 0)),
            pl.BlockSpec((BLK, 1), lambda i: (jnp.maximum(i - 1, 0), 0)),
            pl.BlockSpec((BLK, 1), lambda i: (jnp.maximum(i - 1, 0), 0)),
            pl.BlockSpec((BLK, 1), lambda i: (jnp.maximum(i - 1, 0), 0)),
        ],
        out_shape=[
            jax.ShapeDtypeStruct((n, D), jnp.bfloat16),
            jax.ShapeDtypeStruct((n, 1), jnp.float32),
            jax.ShapeDtypeStruct((n, 1), jnp.int32),
            jax.ShapeDtypeStruct((n, 1), jnp.float32),
        ],
        scratch_shapes=[pltpu.VMEM((BLK, D), jnp.bfloat16)],
    )(x, wc_pad, Wm, wcp, bcp)

    scale = pl.pallas_call(
        _k2_body,
        out_shape=jax.ShapeDtypeStruct((n, 1), jnp.float32),
    )(ent, y, invn)

    def _blk_idx(i):
        return (jnp.where(i < nblk, i, i - nblk + 1), 0)

    out = pl.pallas_call(
        _k3_body,
        grid=(2 * nblk - 1,),
        in_specs=[
            pl.BlockSpec((BLK, D), _blk_idx),
            pl.BlockSpec((BLK, 1), _blk_idx),
            pl.BlockSpec((BLK, 1), _blk_idx),
        ],
        out_specs=pl.BlockSpec((BLK, LANES),
                               lambda i: (jnp.maximum(i - nblk, 0), 0)),
        out_shape=jax.ShapeDtypeStruct((b, LANES), jnp.float32),
        scratch_shapes=[
            pltpu.VMEM((D, LANES), jnp.float32),
            pltpu.VMEM((D, LANES), jnp.bfloat16),
        ],
    )(s, y, scale)

    return out[:, :NCLS]


# speedup vs baseline: 1.1541x; 1.1541x over previous
import jax
import jax.numpy as jnp
from jax.experimental import pallas as pl
from jax.experimental.pallas import tpu as pltpu

NCLS = 17
FILT_K = 100
BLK = 512
D = 2048
LANES = 128
NEG = -1e30
IMAX = jnp.iinfo(jnp.int32).max


def _k1_body(x_ref, wc_ref, wm_ref, wcp_ref, bcp_ref,
             s_ref, ent_ref, y_ref, invn_ref):
    i = pl.program_id(0)

    @pl.when(i == 0)
    def _():
        s_ref[...] = wc_ref[...].astype(jnp.bfloat16)

    @pl.when(i != 0)
    def _():
        s_ref[...] = jnp.dot(x_ref[...], wm_ref[...],
                             preferred_element_type=jnp.float32
                             ).astype(jnp.bfloat16)

    s = s_ref[...]
    logits = jnp.dot(s, wcp_ref[...].astype(jnp.bfloat16),
                     preferred_element_type=jnp.float32) + bcp_ref[...]
    m = jnp.max(logits, axis=1, keepdims=True)
    z = logits - m
    e = jnp.exp(z)
    se = jnp.sum(e, axis=1, keepdims=True)
    p = e / se
    ls = z - jnp.log(se)
    ent = -jnp.sum(p * ls, axis=1, keepdims=True)
    rows = jax.lax.broadcasted_iota(jnp.int32, (BLK, 1), 0)
    pad_row = jnp.logical_and(i == 0, rows >= NCLS)
    ent_ref[...] = jnp.where(pad_row, jnp.float32(jnp.inf), ent)
    lanes = jax.lax.broadcasted_iota(jnp.int32, (BLK, LANES), 1)
    y_ref[...] = jnp.min(jnp.where(logits == m, lanes, LANES), axis=1,
                         keepdims=True)
    sf = s.astype(jnp.float32)
    rn = jnp.sqrt(jnp.sum(sf * sf, axis=1, keepdims=True))
    invn_ref[...] = 1.0 / jnp.maximum(rn, 1e-12)


def _k2_body(ent_ref, y_ref, invn_ref, sc_ref):
    n = ent_ref.shape[0]
    key = jax.lax.bitcast_convert_type(jnp.maximum(ent_ref[...], 0.0),
                                       jnp.int32)
    lanes = jax.lax.broadcasted_iota(jnp.int32, (n, LANES), 1)
    yoh = y_ref[...] == lanes
    mk = jnp.where(yoh, key, IMAX)

    ones_row = jnp.ones((1, n), jnp.float32)

    def body(it, lo):
        mid = lo + jnp.left_shift(jnp.int32(1), 30 - it)
        ltf = jnp.where(mk < mid, 1.0, 0.0)
        cnt = jax.lax.dot_general(ones_row, ltf, (((1,), (0,)), ((), ())),
                                  preferred_element_type=jnp.float32)
        return jnp.where(cnt >= FILT_K, lo, mid)

    lo = jax.lax.fori_loop(0, 31, body, jnp.zeros((1, LANES), jnp.int32))
    tau = jnp.sum(jnp.where(yoh, lo, 0), axis=1, keepdims=True)
    sc_ref[...] = jnp.where(key <= tau, invn_ref[...], 0.0)


def _k3_body(s_ref, y_ref, sc_ref, o_ref, w_ref, wbf_ref):
    i = pl.program_id(0)
    nacc = (pl.num_programs(0) + 1) // 2

    @pl.when(i == 0)
    def _():
        w_ref[...] = jnp.zeros_like(w_ref)

    @pl.when(i < nacc)
    def _():
        lanes = jax.lax.broadcasted_iota(jnp.int32, (BLK, LANES), 1)
        msel = jnp.where(y_ref[...] == lanes, sc_ref[...],
                         0.0).astype(jnp.bfloat16)
        w_ref[...] += jax.lax.dot_general(
            s_ref[...], msel, (((0,), (0,)), ((), ())),
            preferred_element_type=jnp.float32)

    @pl.when(i == nacc - 1)
    def _():
        w = w_ref[...]
        cn = jnp.sqrt(jnp.sum(w * w, axis=0, keepdims=True))
        wbf_ref[...] = (w / jnp.maximum(cn, 1e-12)).astype(jnp.bfloat16)

    @pl.when(i >= nacc)
    def _():
        o_ref[...] = jnp.dot(s_ref[...], wbf_ref[...],
                             preferred_element_type=jnp.float32)


def kernel(x, Wm, Wc, bc):
    b = x.shape[0]
    n = b + BLK
    nblk = n // BLK
    wc_pad = jnp.zeros((BLK, D), jnp.float32).at[:NCLS].set(Wc)
    wcp = jnp.zeros((D, LANES), jnp.float32).at[:, :NCLS].set(Wc.T)
    bcp = jnp.full((1, LANES), NEG, jnp.float32).at[0, :NCLS].set(bc)

    s, ent, y, invn = pl.pallas_call(
        _k1_body,
        grid=(nblk,),
        in_specs=[
            pl.BlockSpec((BLK, D), lambda i: (jnp.maximum(i - 1, 0), 0)),
            pl.BlockSpec((BLK, D), lambda i: (0, 0)),
            pl.BlockSpec((D, D), lambda i: (0, 0)),
            pl.BlockSpec((D, LANES), lambda i: (0, 0)),
            pl.BlockSpec((1, LANES), lambda i: (0, 0)),
        ],
        out_specs=[
            pl.BlockSpec((BLK, D), lambda i: (i, 0)),
            pl.BlockSpec((BLK, 1), lambda i: (i, 0)),
            pl.BlockSpec((BLK, 1), lambda i: (i, 0)),
            pl.BlockSpec((BLK, 1), lambda i: (i, 0)),
        ],
        out_shape=[
            jax.ShapeDtypeStruct((n, D), jnp.bfloat16),
            jax.ShapeDtypeStruct((n, 1), jnp.float32),
            jax.ShapeDtypeStruct((n, 1), jnp.int32),
            jax.ShapeDtypeStruct((n, 1), jnp.float32),
        ],
    )(x, wc_pad, Wm, wcp, bcp)

    scale = pl.pallas_call(
        _k2_body,
        out_shape=jax.ShapeDtypeStruct((n, 1), jnp.float32),
    )(ent, y, invn)

    def _blk_idx(i):
        return (jnp.where(i < nblk, i, i - nblk + 1), 0)

    out = pl.pallas_call(
        _k3_body,
        grid=(2 * nblk - 1,),
        in_specs=[
            pl.BlockSpec((BLK, D), _blk_idx),
            pl.BlockSpec((BLK, 1), _blk_idx),
            pl.BlockSpec((BLK, 1), _blk_idx),
        ],
        out_specs=pl.BlockSpec((BLK, LANES),
                               lambda i: (jnp.maximum(i - nblk, 0), 0)),
        out_shape=jax.ShapeDtypeStruct((b, LANES), jnp.float32),
        scratch_shapes=[
            pltpu.VMEM((D, LANES), jnp.float32),
            pltpu.VMEM((D, LANES), jnp.bfloat16),
        ],
    )(s, y, scale)

    return out[:, :NCLS]
